# speedup vs baseline: 1.0000x; 1.0000x over previous
import jax
import jax.numpy as jnp
from jax.experimental import pallas as pl

_N = 10000
_B = 64
_EPS = 1e-5


def _conv(x, W, b, src, dst):
    h = x @ W
    ones = jnp.ones(src.shape[0], dtype=h.dtype)
    deg = jax.ops.segment_sum(ones, dst, num_segments=_N)
    dinv = jnp.where(deg > 0, 1.0 / jnp.sqrt(deg), 0.0)
    norm = dinv[src] * dinv[dst]
    msg = h[src] * norm[:, None]
    out = jax.ops.segment_sum(msg, dst, num_segments=_N)
    return out + b


def _gnorm(x, gamma, beta, alpha, batch, cnt):
    mean = jax.ops.segment_sum(x, batch, num_segments=_B) / cnt[:, None]
    xc = x - alpha * mean[batch]
    var = jax.ops.segment_sum(xc * xc, batch, num_segments=_B) / cnt[:, None]
    return gamma * xc / jnp.sqrt(var[batch] + _EPS) + beta


def kernel(x, edge_index, batch, W1, b1, g1, be1, a1, W2, b2, g2, be2, a2, W3, b3, g3, be3, a3, Wo, bo):
    loops = jnp.arange(_N, dtype=edge_index.dtype)
    src = jnp.concatenate([edge_index[0], loops])
    dst = jnp.concatenate([edge_index[1], loops])
    cnt = jnp.maximum(jax.ops.segment_sum(jnp.ones((_N,), jnp.float32), batch, num_segments=_B), 1.0)
    h = x
    for (W, b, g, be, a) in ((W1, b1, g1, be1, a1), (W2, b2, g2, be2, a2), (W3, b3, g3, be3, a3)):
        h = _conv(h, W, b, src, dst)
        h = _gnorm(h, g, be, a, batch, cnt)
    pooled = jax.ops.segment_sum(h, batch, num_segments=_B) / cnt[:, None]
    return pooled @ Wo + bo


# speedup vs baseline: 1.5147x; 1.5147x over previous
import functools

import jax
import jax.numpy as jnp
from jax import lax
from jax.experimental import pallas as pl
from jax.experimental.pallas import tpu as pltpu
from jax.experimental.pallas import tpu_sc as plsc

_N = 10000
_NP = 10240
_B = 64
_D = 256
_EPS = 1e-5
_BLK = 1000
_E = 170000
_EP = 172032
_RPT = 320
_QBLK = 2048
_GSUB = 64

_mesh = plsc.VectorSubcoreMesh(core_axis_name="c", subcore_axis_name="s")


def _al(x):
    return pl.multiple_of(x, 8)


def _mm_body(x_ref, w_ref, o_ref):
    o_ref[...] = jnp.dot(x_ref[...], w_ref[...], preferred_element_type=jnp.float32)


def _matmul(x, w):
    n = x.shape[0]
    return pl.pallas_call(
        _mm_body,
        grid=(n // _BLK,),
        in_specs=[
            pl.BlockSpec((_BLK, _D), lambda i: (i, 0)),
            pl.BlockSpec((_D, _D), lambda i: (0, 0)),
        ],
        out_specs=pl.BlockSpec((_BLK, _D), lambda i: (i, 0)),
        out_shape=jax.ShapeDtypeStruct((n, _D), jnp.float32),
    )(x, w)


def _head_body(p_ref, w_ref, b_ref, o_ref):
    o_ref[...] = jnp.dot(p_ref[...], w_ref[...], preferred_element_type=jnp.float32) + b_ref[...]


def _head(pooled, wo_pad, bo_pad):
    return pl.pallas_call(
        _head_body,
        out_shape=jax.ShapeDtypeStruct((_B, 128), jnp.float32),
    )(pooled, wo_pad, bo_pad)


def _dinv_body(h_ref, o_ref):
    deg = jnp.sum(h_ref[...], axis=1, keepdims=True) / 16.0
    dinv = jnp.where(deg > 0, lax.rsqrt(deg), 0.0)
    o_ref[...] = jnp.broadcast_to(dinv, (_NP, 128))


def _dinv_tc(deg16):
    return pl.pallas_call(
        _dinv_body,
        out_shape=jax.ShapeDtypeStruct((_NP, 128), jnp.float32),
    )(deg16)


def _deg_body(dstq_hbm, ts_hbm, te_hbm, deg16_hbm, dbuf, tb, hist, sem):
    wid = lax.axis_index("s") * 2 + lax.axis_index("c")
    lo = wid * _RPT
    zero16 = jnp.zeros((16,), jnp.float32)

    def zloop(i, _):
        hist[pl.ds(_al(i * 16), 16)] = zero16
        return 0
    lax.fori_loop(0, _RPT, zloop, 0)

    pltpu.sync_copy(ts_hbm.at[pl.ds(_al(wid * 16), 16)], tb)
    start = tb[pl.ds(0, 16)][0]
    pltpu.sync_copy(te_hbm.at[pl.ds(_al(wid * 16), 16)], tb)
    end = tb[pl.ds(0, 16)][0]

    s8 = _al((start // 8) * 8)
    nblk = (end - s8 + (_QBLK - 1)) // _QBLK

    def blk(kb, _):
        base = _al(s8 + kb * _QBLK)
        pltpu.sync_copy(dstq_hbm.at[pl.ds(base, _QBLK)], dbuf)

        def chunk(c, _):
            dlv = dbuf[pl.ds(_al(c * 16), 16)] - lo
            for k in range(16):
                e = base + c * 16 + k

                @pl.when((e >= start) & (e < end))
                def _():
                    off = _al(dlv[k] * 16)
                    hist[pl.ds(off, 16)] = hist[pl.ds(off, 16)] + 1.0
            return 0
        lax.fori_loop(0, _QBLK // 16, chunk, 0)
        return 0
    lax.fori_loop(0, nblk, blk, 0)

    pltpu.sync_copy(hist, deg16_hbm.at[pl.ds(_al(lo * 16), _RPT * 16)])


@functools.partial(
    pl.kernel,
    out_type=jax.ShapeDtypeStruct((_NP * 16,), jnp.float32),
    mesh=_mesh,
    scratch_types=[
        pltpu.VMEM((_QBLK,), jnp.int32),
        pltpu.VMEM((16,), jnp.int32),
        pltpu.VMEM((_RPT * 16,), jnp.float32),
        pltpu.SemaphoreType.DMA,
    ],
)
def _sc_deg(dstq_hbm, ts_hbm, te_hbm, deg16_hbm, dbuf, tb, hist, sem):
    _deg_body(dstq_hbm, ts_hbm, te_hbm, deg16_hbm, dbuf, tb, hist, sem)


def _agg_body(h_hbm, dinv16_hbm, srcq_hbm, dstq_hbm, ts_hbm, te_hbm, b_hbm, z_hbm,
              out_hbm, qs, qd, tb, gbuf, dsb, ddb, acc, b_v, sem):
    wid = lax.axis_index("s") * 2 + lax.axis_index("c")
    lo = wid * _RPT

    pltpu.sync_copy(b_hbm, b_v)
    pltpu.sync_copy(z_hbm, acc)
    pltpu.sync_copy(ts_hbm.at[pl.ds(_al(wid * 16), 16)], tb)
    start = tb[pl.ds(0, 16)][0]
    pltpu.sync_copy(te_hbm.at[pl.ds(_al(wid * 16), 16)], tb)
    end = tb[pl.ds(0, 16)][0]

    s8 = _al((start // 8) * 8)
    nblk = (end - s8 + (_QBLK - 1)) // _QBLK

    def blk(kb, _):
        base = _al(s8 + kb * _QBLK)
        pltpu.sync_copy(srcq_hbm.at[pl.ds(base, _QBLK)], qs)
        pltpu.sync_copy(dstq_hbm.at[pl.ds(base, _QBLK)], qd)

        def sub(j, _):
            idx = qs.at[pl.ds(_al(j * _GSUB), _GSUB)]
            idxd = qd.at[pl.ds(_al(j * _GSUB), _GSUB)]
            pltpu.async_copy(h_hbm.at[idx], gbuf, sem).wait()
            pltpu.async_copy(dinv16_hbm.at[idx], dsb, sem).wait()
            pltpu.async_copy(dinv16_hbm.at[idxd], ddb, sem).wait()
            e_lo = j * _GSUB

            def chunk(c, _):
                qdv = qd[pl.ds(_al(e_lo + c * 16), 16)] - lo
                for k in range(16):
                    e = base + e_lo + c * 16 + k

                    @pl.when((e >= start) & (e < end))
                    def _():
                        re = c * 16 + k
                        nrm = dsb[re, pl.ds(0, 16)][0] * ddb[re, pl.ds(0, 16)][0]
                        aoff = _al(qdv[k] * _D)
                        for t in range(16):
                            a16 = acc[pl.ds(aoff + t * 16, 16)]
                            g16 = gbuf[re, pl.ds(t * 16, 16)]
                            acc[pl.ds(aoff + t * 16, 16)] = a16 + g16 * nrm
                return 0
            lax.fori_loop(0, _GSUB // 16, chunk, 0)
            return 0
        lax.fori_loop(0, _QBLK // _GSUB, sub, 0)
        return 0
    lax.fori_loop(0, nblk, blk, 0)

    def brow(r, _):
        roff = _al(r * _D)
        for k in range(16):
            acc[pl.ds(roff + k * 16, 16)] = (acc[pl.ds(roff + k * 16, 16)]
                                             + b_v[pl.ds(k * 16, 16)])
        return 0
    lax.fori_loop(0, _RPT, brow, 0)

    pltpu.sync_copy(acc, out_hbm.at[pl.ds(_al(lo * _D), _RPT * _D)])


@functools.partial(
    pl.kernel,
    out_type=jax.ShapeDtypeStruct((_NP * _D,), jnp.float32),
    mesh=_mesh,
    scratch_types=[
        pltpu.VMEM((_QBLK,), jnp.int32),
        pltpu.VMEM((_QBLK,), jnp.int32),
        pltpu.VMEM((16,), jnp.int32),
        pltpu.VMEM((_GSUB, _D), jnp.float32),
        pltpu.VMEM((_GSUB, 128), jnp.float32),
        pltpu.VMEM((_GSUB, 128), jnp.float32),
        pltpu.VMEM((_RPT * _D,), jnp.float32),
        pltpu.VMEM((_D,), jnp.float32),
        pltpu.SemaphoreType.DMA,
    ],
)
def _sc_agg(h_hbm, dinv16_hbm, srcq_hbm, dstq_hbm, ts_hbm, te_hbm, b_hbm, z_hbm,
            out_hbm, qs, qd, tb, gbuf, dsb, ddb, acc, b_v, sem):
    _agg_body(h_hbm, dinv16_hbm, srcq_hbm, dstq_hbm, ts_hbm, te_hbm, b_hbm, z_hbm,
              out_hbm, qs, qd, tb, gbuf, dsb, ddb, acc, b_v, sem)


def kernel(x, edge_index, batch, W1, b1, g1, be1, a1, W2, b2, g2, be2, a2, W3, b3, g3, be3, a3, Wo, bo):
    loops = jnp.arange(_N, dtype=edge_index.dtype)
    src = jnp.concatenate([edge_index[0], loops])
    dst = jnp.concatenate([edge_index[1], loops])
    cnt = jnp.maximum(jax.ops.segment_sum(jnp.ones((_N,), jnp.float32), batch, num_segments=_B), 1.0)

    order = jnp.argsort(dst, stable=True)
    srcq = jnp.zeros((_EP,), jnp.int32).at[: _E].set(src[order])
    dstq = jnp.zeros((_EP,), jnp.int32).at[: _E].set(dst[order])
    starts = jnp.searchsorted(dstq[: _E], jnp.arange(33, dtype=jnp.int32) * _RPT).astype(jnp.int32)
    ts16 = jnp.repeat(starts[:32], 16)
    te16 = jnp.repeat(starts[1:33], 16)

    deg16 = _sc_deg(dstq, ts16, te16)
    dinv16 = _dinv_tc(deg16.reshape(_NP, 16))

    zflat = jnp.zeros((_RPT * _D,), jnp.float32)

    h = x
    for (W, b, g, be, a) in ((W1, b1, g1, be1, a1), (W2, b2, g2, be2, a2), (W3, b3, g3, be3, a3)):
        hw = _matmul(h, W)
        out1 = _sc_agg(hw, dinv16, srcq, dstq, ts16, te16, b, zflat).reshape(_NP, _D)[:_N]
        mean = jax.ops.segment_sum(out1, batch, num_segments=_B) / cnt[:, None]
        xc = out1 - a * mean[batch]
        var = jax.ops.segment_sum(xc * xc, batch, num_segments=_B) / cnt[:, None]
        h = g * xc / jnp.sqrt(var[batch] + _EPS) + be

    pooled = jax.ops.segment_sum(h, batch, num_segments=_B) / cnt[:, None]
    wo_pad = jnp.pad(Wo, ((0, 0), (0, 126)))
    bo_pad = jnp.pad(bo, (0, 126)).reshape(1, 128)
    return _head(pooled, wo_pad, bo_pad)[:, :2]


# speedup vs baseline: 1.6140x; 1.0655x over previous
import functools

import jax
import jax.numpy as jnp
from jax import lax
from jax.experimental import pallas as pl
from jax.experimental.pallas import tpu as pltpu
from jax.experimental.pallas import tpu_sc as plsc

_N = 10000
_NP = 10240
_B = 64
_D = 256
_EPS = 1e-5
_BLK = 1000
_E = 170000
_EP = 172032
_RPT = 320
_QBLK = 2048
_GSUB = 64

_mesh = plsc.VectorSubcoreMesh(core_axis_name="c", subcore_axis_name="s")


def _al(x):
    return pl.multiple_of(x, 8)


def _mm_body(x_ref, w_ref, o_ref):
    o_ref[...] = jnp.dot(x_ref[...], w_ref[...], preferred_element_type=jnp.float32)


def _matmul(x, w):
    n = x.shape[0]
    return pl.pallas_call(
        _mm_body,
        grid=(n // _BLK,),
        in_specs=[
            pl.BlockSpec((_BLK, _D), lambda i: (i, 0)),
            pl.BlockSpec((_D, _D), lambda i: (0, 0)),
        ],
        out_specs=pl.BlockSpec((_BLK, _D), lambda i: (i, 0)),
        out_shape=jax.ShapeDtypeStruct((n, _D), jnp.float32),
    )(x, w)


def _head_body(p_ref, w_ref, b_ref, o_ref):
    o_ref[...] = jnp.dot(p_ref[...], w_ref[...], preferred_element_type=jnp.float32) + b_ref[...]


def _head(pooled, wo_pad, bo_pad):
    return pl.pallas_call(
        _head_body,
        out_shape=jax.ShapeDtypeStruct((_B, 128), jnp.float32),
    )(pooled, wo_pad, bo_pad)


def _dinv_body(h_ref, o_ref):
    deg = jnp.sum(h_ref[...], axis=1, keepdims=True) / 16.0
    dinv = jnp.where(deg > 0, lax.rsqrt(deg), 0.0)
    o_ref[...] = jnp.broadcast_to(dinv, (_NP, 128))


def _dinv_tc(deg16):
    return pl.pallas_call(
        _dinv_body,
        out_shape=jax.ShapeDtypeStruct((_NP, 128), jnp.float32),
    )(deg16)


def _deg_body(dstq_hbm, ts_hbm, te_hbm, deg16_hbm, dbuf, tb, hist, sem):
    wid = lax.axis_index("s") * 2 + lax.axis_index("c")
    lo = wid * _RPT
    zero16 = jnp.zeros((16,), jnp.float32)

    def zloop(i, _):
        hist[pl.ds(_al(i * 16), 16)] = zero16
        return 0
    lax.fori_loop(0, _RPT, zloop, 0)

    pltpu.sync_copy(ts_hbm.at[pl.ds(_al(wid * 16), 16)], tb)
    start = tb[pl.ds(0, 16)][0]
    pltpu.sync_copy(te_hbm.at[pl.ds(_al(wid * 16), 16)], tb)
    end = tb[pl.ds(0, 16)][0]

    s8 = _al((start // 8) * 8)
    nblk = (end - s8 + (_QBLK - 1)) // _QBLK

    def blk(kb, _):
        base = _al(s8 + kb * _QBLK)
        pltpu.sync_copy(dstq_hbm.at[pl.ds(base, _QBLK)], dbuf)

        def chunk(c, _):
            dlv = dbuf[pl.ds(_al(c * 16), 16)] - lo
            for k in range(16):
                e = base + c * 16 + k

                @pl.when((e >= start) & (e < end))
                def _():
                    off = _al(dlv[k] * 16)
                    hist[pl.ds(off, 16)] = hist[pl.ds(off, 16)] + 1.0
            return 0
        lax.fori_loop(0, _QBLK // 16, chunk, 0)
        return 0
    lax.fori_loop(0, nblk, blk, 0)

    pltpu.sync_copy(hist, deg16_hbm.at[pl.ds(_al(lo * 16), _RPT * 16)])


@functools.partial(
    pl.kernel,
    out_type=jax.ShapeDtypeStruct((_NP * 16,), jnp.float32),
    mesh=_mesh,
    scratch_types=[
        pltpu.VMEM((_QBLK,), jnp.int32),
        pltpu.VMEM((16,), jnp.int32),
        pltpu.VMEM((_RPT * 16,), jnp.float32),
        pltpu.SemaphoreType.DMA,
    ],
)
def _sc_deg(dstq_hbm, ts_hbm, te_hbm, deg16_hbm, dbuf, tb, hist, sem):
    _deg_body(dstq_hbm, ts_hbm, te_hbm, deg16_hbm, dbuf, tb, hist, sem)


def _agg_body(h_hbm, dinv16_hbm, srcq_hbm, dstq_hbm, ts_hbm, te_hbm, b_hbm, z_hbm,
              out_hbm, qs, qd, tb, gbuf, dsb, ddb, acc, b_v, sem, sem2, sem3):
    wid = lax.axis_index("s") * 2 + lax.axis_index("c")
    lo = wid * _RPT

    pltpu.sync_copy(b_hbm, b_v)
    pltpu.sync_copy(z_hbm, acc)
    pltpu.sync_copy(ts_hbm.at[pl.ds(_al(wid * 16), 16)], tb)
    start = tb[pl.ds(0, 16)][0]
    pltpu.sync_copy(te_hbm.at[pl.ds(_al(wid * 16), 16)], tb)
    end = tb[pl.ds(0, 16)][0]

    s8 = _al((start // 8) * 8)
    nblk = (end - s8 + (_QBLK - 1)) // _QBLK

    def blk(kb, _):
        base = _al(s8 + kb * _QBLK)
        pltpu.sync_copy(srcq_hbm.at[pl.ds(base, _QBLK)], qs)
        pltpu.sync_copy(dstq_hbm.at[pl.ds(base, _QBLK)], qd)

        def sub(j, _):
            idx = qs.at[pl.ds(_al(j * _GSUB), _GSUB)]
            idxd = qd.at[pl.ds(_al(j * _GSUB), _GSUB)]
            cp1 = pltpu.async_copy(h_hbm.at[idx], gbuf, sem)
            cp2 = pltpu.async_copy(dinv16_hbm.at[idx], dsb, sem2)
            cp3 = pltpu.async_copy(dinv16_hbm.at[idxd], ddb, sem3)
            cp1.wait()
            cp2.wait()
            cp3.wait()
            e_lo = j * _GSUB

            def chunk(c, _):
                qdv = qd[pl.ds(_al(e_lo + c * 16), 16)] - lo
                for k in range(16):
                    e = base + e_lo + c * 16 + k

                    @pl.when((e >= start) & (e < end))
                    def _():
                        re = c * 16 + k
                        nrm = dsb[re, pl.ds(0, 16)][0] * ddb[re, pl.ds(0, 16)][0]
                        aoff = _al(qdv[k] * _D)
                        for t in range(16):
                            a16 = acc[pl.ds(aoff + t * 16, 16)]
                            g16 = gbuf[re, pl.ds(t * 16, 16)]
                            acc[pl.ds(aoff + t * 16, 16)] = a16 + g16 * nrm
                return 0
            lax.fori_loop(0, _GSUB // 16, chunk, 0)
            return 0
        lax.fori_loop(0, _QBLK // _GSUB, sub, 0)
        return 0
    lax.fori_loop(0, nblk, blk, 0)

    def brow(r, _):
        roff = _al(r * _D)
        for k in range(16):
            acc[pl.ds(roff + k * 16, 16)] = (acc[pl.ds(roff + k * 16, 16)]
                                             + b_v[pl.ds(k * 16, 16)])
        return 0
    lax.fori_loop(0, _RPT, brow, 0)

    pltpu.sync_copy(acc, out_hbm.at[pl.ds(_al(lo * _D), _RPT * _D)])


@functools.partial(
    pl.kernel,
    out_type=jax.ShapeDtypeStruct((_NP * _D,), jnp.float32),
    mesh=_mesh,
    scratch_types=[
        pltpu.VMEM((_QBLK,), jnp.int32),
        pltpu.VMEM((_QBLK,), jnp.int32),
        pltpu.VMEM((16,), jnp.int32),
        pltpu.VMEM((_GSUB, _D), jnp.float32),
        pltpu.VMEM((_GSUB, 128), jnp.float32),
        pltpu.VMEM((_GSUB, 128), jnp.float32),
        pltpu.VMEM((_RPT * _D,), jnp.float32),
        pltpu.VMEM((_D,), jnp.float32),
        pltpu.SemaphoreType.DMA,
        pltpu.SemaphoreType.DMA,
        pltpu.SemaphoreType.DMA,
    ],
)
def _sc_agg(h_hbm, dinv16_hbm, srcq_hbm, dstq_hbm, ts_hbm, te_hbm, b_hbm, z_hbm,
            out_hbm, qs, qd, tb, gbuf, dsb, ddb, acc, b_v, sem, sem2, sem3):
    _agg_body(h_hbm, dinv16_hbm, srcq_hbm, dstq_hbm, ts_hbm, te_hbm, b_hbm, z_hbm,
              out_hbm, qs, qd, tb, gbuf, dsb, ddb, acc, b_v, sem, sem2, sem3)


def kernel(x, edge_index, batch, W1, b1, g1, be1, a1, W2, b2, g2, be2, a2, W3, b3, g3, be3, a3, Wo, bo):
    loops = jnp.arange(_N, dtype=edge_index.dtype)
    src = jnp.concatenate([edge_index[0], loops])
    dst = jnp.concatenate([edge_index[1], loops])
    cnt = jnp.maximum(jax.ops.segment_sum(jnp.ones((_N,), jnp.float32), batch, num_segments=_B), 1.0)

    order = jnp.argsort(dst, stable=True)
    srcq = jnp.zeros((_EP,), jnp.int32).at[: _E].set(src[order])
    dstq = jnp.zeros((_EP,), jnp.int32).at[: _E].set(dst[order])
    starts = jnp.searchsorted(dstq[: _E], jnp.arange(33, dtype=jnp.int32) * _RPT).astype(jnp.int32)
    ts16 = jnp.repeat(starts[:32], 16)
    te16 = jnp.repeat(starts[1:33], 16)

    deg16 = _sc_deg(dstq, ts16, te16)
    dinv16 = _dinv_tc(deg16.reshape(_NP, 16))

    zflat = jnp.zeros((_RPT * _D,), jnp.float32)

    h = x
    for (W, b, g, be, a) in ((W1, b1, g1, be1, a1), (W2, b2, g2, be2, a2), (W3, b3, g3, be3, a3)):
        hw = _matmul(h, W)
        out1 = _sc_agg(hw, dinv16, srcq, dstq, ts16, te16, b, zflat).reshape(_NP, _D)[:_N]
        mean = jax.ops.segment_sum(out1, batch, num_segments=_B) / cnt[:, None]
        xc = out1 - a * mean[batch]
        var = jax.ops.segment_sum(xc * xc, batch, num_segments=_B) / cnt[:, None]
        h = g * xc / jnp.sqrt(var[batch] + _EPS) + be

    pooled = jax.ops.segment_sum(h, batch, num_segments=_B) / cnt[:, None]
    wo_pad = jnp.pad(Wo, ((0, 0), (0, 126)))
    bo_pad = jnp.pad(bo, (0, 126)).reshape(1, 128)
    return _head(pooled, wo_pad, bo_pad)[:, :2]
